# baseline (speedup 1.0000x reference)
import functools

import jax
import jax.numpy as jnp
from jax import lax
from jax.experimental import pallas as pl
from jax.experimental.pallas import tpu as pltpu
from jax.experimental.pallas import tpu_sc as plsc

_D = 768
_BATCH = 4
_SEQ = 2048
_NC = 2
_NS = 16
_NW = _NC * _NS
_PP = _SEQ // _NW
_LPT = _D // 16


@functools.partial(
    pl.kernel,
    out_type=jax.ShapeDtypeStruct((_BATCH * _SEQ, _D), jnp.float32),
    mesh=plsc.VectorSubcoreMesh(core_axis_name="c", subcore_axis_name="s"),
    scratch_types=[
        pltpu.VMEM((_PP,), jnp.int32),
        pltpu.VMEM((_PP, _D), jnp.float32),
        pltpu.VMEM((_PP, _D), jnp.float32),
        pltpu.SemaphoreType.DMA,
    ],
)
def _emb_kernel(x_hbm, wte_hbm, wpe_hbm, out_hbm, idx_v, wpe_v, rows_v, sem):
    wid = lax.axis_index("s") * _NC + lax.axis_index("c")
    pos0 = wid * _PP
    pltpu.sync_copy(wpe_hbm.at[pl.ds(pos0, _PP)], wpe_v)
    for b in range(_BATCH):
        pltpu.sync_copy(x_hbm.at[pl.ds(b * _SEQ + pos0, _PP)], idx_v)
        pltpu.async_copy(wte_hbm.at[idx_v], rows_v, sem).wait()

        def tok_body(t, carry):
            for dd in range(_LPT):
                sl = pl.ds(dd * 16, 16)
                rows_v[t, sl] = rows_v[t, sl] + wpe_v[t, sl]
            return carry

        lax.fori_loop(0, _PP, tok_body, 0)
        pltpu.sync_copy(rows_v, out_hbm.at[pl.ds(b * _SEQ + pos0, _PP)])


def kernel(X, wte, wpe):
    xf = X.reshape(-1).astype(jnp.int32)
    out = _emb_kernel(xf, wte, wpe)
    return out.reshape(_BATCH, _SEQ, _D)


# speedup vs baseline: 1.2481x; 1.2481x over previous
import functools

import jax
import jax.numpy as jnp
from jax import lax
from jax.experimental import pallas as pl
from jax.experimental.pallas import tpu as pltpu
from jax.experimental.pallas import tpu_sc as plsc

_D = 768
_BATCH = 4
_SEQ = 2048
_NC = 2
_NS = 16
_NW = _NC * _NS
_PP = _SEQ // _NW
_C = 32
_NK = _PP // _C
_NSTEP = _NK * _BATCH
_LPT = _D // 16


@functools.partial(
    pl.kernel,
    out_type=jax.ShapeDtypeStruct((_BATCH * _SEQ, _D), jnp.float32),
    mesh=plsc.VectorSubcoreMesh(core_axis_name="c", subcore_axis_name="s"),
    scratch_types=[
        pltpu.VMEM((_NSTEP, _C), jnp.int32),
        [pltpu.VMEM((_C, _D), jnp.float32) for _ in range(3)],
        [pltpu.VMEM((_C, _D), jnp.float32) for _ in range(2)],
        [pltpu.SemaphoreType.DMA for _ in range(3)],
        [pltpu.SemaphoreType.DMA for _ in range(3)],
        pltpu.SemaphoreType.DMA,
        pltpu.SemaphoreType.DMA,
    ],
)
def _emb_kernel(x_hbm, wte_hbm, wpe_hbm, out_hbm,
                idx_v, rows, wpes, gsem, ssem, isem, wsem):
    wid = lax.axis_index("s") * _NC + lax.axis_index("c")
    pos0 = wid * _PP

    def row_off(i):
        k, b = divmod(i, _BATCH)
        return b * _SEQ + pos0 + k * _C

    idx_cp = [
        pltpu.async_copy(x_hbm.at[pl.ds(row_off(i), _C)], idx_v.at[i], isem)
        for i in range(_NSTEP)
    ]
    for cp in idx_cp:
        cp.wait()

    pltpu.sync_copy(wpe_hbm.at[pl.ds(pos0, _C)], wpes[0])
    wpe_cp = pltpu.async_copy(wpe_hbm.at[pl.ds(pos0 + _C, _C)], wpes[1], wsem)

    def gather(i):
        return pltpu.async_copy(wte_hbm.at[idx_v.at[i]], rows[i % 3],
                                gsem[i % 3])

    g_cp = {0: gather(0), 1: gather(1)}
    s_cp = {}
    for i in range(_NSTEP):
        k = i // _BATCH
        if i == _BATCH:
            wpe_cp.wait()
        g_cp[i].wait()
        buf, wpe_b = rows[i % 3], wpes[k]

        def tok_body(t, carry):
            for dd in range(_LPT):
                sl = pl.ds(dd * 16, 16)
                buf[t, sl] = buf[t, sl] + wpe_b[t, sl]
            return carry

        lax.fori_loop(0, _C, tok_body, 0)
        if i + 2 < _NSTEP:
            if i - 1 >= 0:
                s_cp[i - 1].wait()
            g_cp[i + 2] = gather(i + 2)
        s_cp[i] = pltpu.async_copy(
            buf, out_hbm.at[pl.ds(row_off(i), _C)], ssem[i % 3])
    for i in range(_NSTEP - 3, _NSTEP):
        s_cp[i].wait()


def kernel(X, wte, wpe):
    xf = X.reshape(-1).astype(jnp.int32)
    out = _emb_kernel(xf, wte, wpe)
    return out.reshape(_BATCH, _SEQ, _D)
